# baseline (device time: 55425 ns/iter reference)
import jax
import jax.numpy as jnp
from jax import lax
from jax.experimental import pallas as pl
from jax.experimental.pallas import tpu as pltpu

N_DEV = 16
N_TOK = 2048
D_IN = 512
D_OUT = 1024
H_HALF = D_OUT // 2
E_LOCAL = 4
CHUNK = N_TOK // N_DEV
GROUP = 4 * CHUNK

WIRE_DTYPE = jnp.bfloat16


def kernel(x, router_W, route_idx, expert_W):
    ew2 = expert_W.reshape(E_LOCAL * D_IN, D_OUT)

    def body(x_ref, rw_ref, idx_ref, ew_ref, out_ref,
             gates_ref, ewb_ref, par_ref, pal_ref,
             stap_ref, stam_ref, stbp_ref, stbm_ref,
             rap_ref, ram_ref, rbp_ref, rbm_ref,
             sap_sem, sam_sem, sbp_sem, sbm_sem,
             rap_sems, ram_sems, rbp_sems, rbm_sems):
        my = lax.axis_index("i")
        k = lax.rem(my, 4)
        z = lax.div(my, 4)

        def m4(v):
            return lax.rem(v + 8, 4)

        kp1 = z * 4 + m4(k + 1)
        km1 = z * 4 + m4(k - 1)
        zp1 = m4(z + 1) * 4 + k
        zm1 = m4(z - 1) * 4 + k

        barrier = pltpu.get_barrier_semaphore()
        for nbr in (kp1, km1, zp1, zm1):
            pl.semaphore_signal(barrier, inc=1, device_id=(nbr,),
                                device_id_type=pl.DeviceIdType.MESH)
        pl.semaphore_wait(barrier, 4)

        scores = jnp.dot(x_ref[:, :], rw_ref[:, :],
                         preferred_element_type=jnp.float32)
        r0 = idx_ref[:, 0:1]
        r1 = idx_ref[:, 1:2]
        eids = lax.broadcasted_iota(jnp.int32, (1, 64), 1)
        s0 = jnp.sum(scores * (r0 == eids).astype(jnp.float32),
                     axis=1, keepdims=True)
        s1 = jnp.sum(scores * (r1 == eids).astype(jnp.float32),
                     axis=1, keepdims=True)
        m = jnp.maximum(s0, s1)
        p0 = jnp.exp(s0 - m)
        p1 = jnp.exp(s1 - m)
        g0 = p0 / (p0 + p1)
        g1 = p1 / (p0 + p1)

        for j in range(E_LOCAL):
            e_glob = my * E_LOCAL + j
            gates_ref[:, j:j + 1] = (jnp.where(r0 == e_glob, g0, 0.0)
                                     + jnp.where(r1 == e_glob, g1, 0.0))

        ewb_ref[:, :] = ew_ref[:, :].astype(WIRE_DTYPE)

        def group_half(g, col0):
            parts = []
            for zp in range(4):
                row0 = (4 * zp + g) * CHUNK
                xc = x_ref[pl.ds(row0, CHUNK), :]
                parts.append(jnp.concatenate(
                    [(xc * gates_ref[pl.ds(row0, CHUNK), j:j + 1])
                     .astype(WIRE_DTYPE) for j in range(E_LOCAL)], axis=1))
            xcat = jnp.concatenate(parts, axis=0)
            return jnp.dot(xcat, ewb_ref[:, col0:col0 + H_HALF],
                           preferred_element_type=jnp.float32)

        def mk(src_ref, dst_ref, ssem, rsem, tgt):
            return pltpu.make_async_remote_copy(
                src_ref=src_ref, dst_ref=dst_ref, send_sem=ssem,
                recv_sem=rsem, device_id=(tgt,),
                device_id_type=pl.DeviceIdType.MESH)

        a_p = [mk(stap_ref, rap_ref.at[s], sap_sem, rap_sems.at[s], kp1)
               for s in range(3)]
        a_m = [mk(stam_ref, ram_ref.at[s], sam_sem, ram_sems.at[s], km1)
               for s in range(3)]
        b_p = [mk(stbp_ref, rbp_ref.at[s], sbp_sem, rbp_sems.at[s], zp1)
               for s in range(3)]
        b_m = [mk(stbm_ref, rbm_ref.at[s], sbm_sem, rbm_sems.at[s], zm1)
               for s in range(3)]

        f32 = jnp.float32

        stap_ref[:, :] = group_half(m4(k - 1), 0).astype(WIRE_DTYPE)
        a_p[0].start()
        stam_ref[:, :] = group_half(m4(k + 1), H_HALF).astype(WIRE_DTYPE)
        a_m[0].start()

        pr1 = group_half(m4(k + 2), 0)
        pl1 = group_half(m4(k + 2), H_HALF)
        a_p[0].wait_recv()
        a_p[0].wait_send()
        stap_ref[:, :] = (rap_ref[0].astype(f32) + pr1).astype(WIRE_DTYPE)
        a_p[1].start()
        a_m[0].wait_recv()
        a_m[0].wait_send()
        stam_ref[:, :] = (ram_ref[0].astype(f32) + pl1).astype(WIRE_DTYPE)
        a_m[1].start()

        pr2 = group_half(m4(k + 1), 0)
        pl2 = group_half(m4(k - 1), H_HALF)
        a_p[1].wait_recv()
        a_p[1].wait_send()
        stap_ref[:, :] = (rap_ref[1].astype(f32) + pr2).astype(WIRE_DTYPE)
        a_p[2].start()
        a_m[1].wait_recv()
        a_m[1].wait_send()
        stam_ref[:, :] = (ram_ref[1].astype(f32) + pl2).astype(WIRE_DTYPE)
        a_m[2].start()

        prf = group_half(k, 0)
        plf = group_half(k, H_HALF)
        a_p[2].wait_recv()
        par_ref[:, :] = rap_ref[2].astype(f32) + prf
        a_m[2].wait_recv()
        pal_ref[:, :] = ram_ref[2].astype(f32) + plf

        stbp_ref[:, :] = par_ref[pl.ds(m4(z - 1) * CHUNK, CHUNK), :] \
            .astype(WIRE_DTYPE)
        b_p[0].start()
        stbm_ref[:, :] = pal_ref[pl.ds(m4(z + 1) * CHUNK, CHUNK), :] \
            .astype(WIRE_DTYPE)
        b_m[0].start()
        for s in range(3):
            b_p[s].wait_recv()
            acc_p = (rbp_ref[s].astype(f32)
                     + par_ref[pl.ds(m4(z - 2 - s) * CHUNK, CHUNK), :])
            if s < 2:
                b_p[s].wait_send()
                stbp_ref[:, :] = acc_p.astype(WIRE_DTYPE)
                b_p[s + 1].start()
            else:
                out_ref[:, 0:H_HALF] = acc_p
            b_m[s].wait_recv()
            acc_m = (rbm_ref[s].astype(f32)
                     + pal_ref[pl.ds(m4(z + 2 + s) * CHUNK, CHUNK), :])
            if s < 2:
                b_m[s].wait_send()
                stbm_ref[:, :] = acc_m.astype(WIRE_DTYPE)
                b_m[s + 1].start()
            else:
                out_ref[:, H_HALF:D_OUT] = acc_m

        a_p[2].wait_send()
        a_m[2].wait_send()
        b_p[2].wait_send()
        b_m[2].wait_send()

    return pl.pallas_call(
        body,
        out_shape=jax.ShapeDtypeStruct((CHUNK, D_OUT), jnp.float32),
        in_specs=[
            pl.BlockSpec(memory_space=pltpu.VMEM),
            pl.BlockSpec(memory_space=pltpu.VMEM),
            pl.BlockSpec(memory_space=pltpu.VMEM),
            pl.BlockSpec(memory_space=pltpu.VMEM),
        ],
        out_specs=pl.BlockSpec(memory_space=pltpu.VMEM),
        scratch_shapes=[
            pltpu.VMEM((N_TOK, E_LOCAL), jnp.float32),
            pltpu.VMEM((E_LOCAL * D_IN, D_OUT), WIRE_DTYPE),
            pltpu.VMEM((GROUP, H_HALF), jnp.float32),
            pltpu.VMEM((GROUP, H_HALF), jnp.float32),
            pltpu.VMEM((GROUP, H_HALF), WIRE_DTYPE),
            pltpu.VMEM((GROUP, H_HALF), WIRE_DTYPE),
            pltpu.VMEM((CHUNK, H_HALF), WIRE_DTYPE),
            pltpu.VMEM((CHUNK, H_HALF), WIRE_DTYPE),
            pltpu.VMEM((3, GROUP, H_HALF), WIRE_DTYPE),
            pltpu.VMEM((3, GROUP, H_HALF), WIRE_DTYPE),
            pltpu.VMEM((3, CHUNK, H_HALF), WIRE_DTYPE),
            pltpu.VMEM((3, CHUNK, H_HALF), WIRE_DTYPE),
            pltpu.SemaphoreType.DMA,
            pltpu.SemaphoreType.DMA,
            pltpu.SemaphoreType.DMA,
            pltpu.SemaphoreType.DMA,
            pltpu.SemaphoreType.DMA((3,)),
            pltpu.SemaphoreType.DMA((3,)),
            pltpu.SemaphoreType.DMA((3,)),
            pltpu.SemaphoreType.DMA((3,)),
        ],
        compiler_params=pltpu.CompilerParams(collective_id=0),
    )(x, router_W, route_idx, ew2)


# device time: 51765 ns/iter; 1.0707x vs baseline; 1.0707x over previous
import jax
import jax.numpy as jnp
from jax import lax
from jax.experimental import pallas as pl
from jax.experimental.pallas import tpu as pltpu

N_DEV = 16
N_TOK = 2048
D_IN = 512
D_OUT = 1024
H_HALF = D_OUT // 2
E_LOCAL = 4
CHUNK = N_TOK // N_DEV
GROUP = 4 * CHUNK

WIRE_DTYPE = jnp.bfloat16


def kernel(x, router_W, route_idx, expert_W):
    ew2 = expert_W.reshape(E_LOCAL * D_IN, D_OUT)

    def body(x_ref, rw_ref, idx_ref, ew_ref, out_ref,
             gates_ref, xb_ref, ewb_ref, par_ref, pal_ref,
             stap_ref, stam_ref, stbp_ref, stbm_ref,
             rap_ref, ram_ref, rbp_ref, rbm_ref,
             sap_sem, sam_sem, sbp_sem, sbm_sem,
             rap_sems, ram_sems, rbp_sems, rbm_sems):
        my = lax.axis_index("i")
        k = lax.rem(my, 4)
        z = lax.div(my, 4)

        def m4(v):
            return lax.rem(v + 8, 4)

        kp1 = z * 4 + m4(k + 1)
        km1 = z * 4 + m4(k - 1)
        zp1 = m4(z + 1) * 4 + k
        zm1 = m4(z - 1) * 4 + k

        barrier = pltpu.get_barrier_semaphore()
        for nbr in (kp1, km1, zp1, zm1):
            pl.semaphore_signal(barrier, inc=1, device_id=(nbr,),
                                device_id_type=pl.DeviceIdType.MESH)
        pl.semaphore_wait(barrier, 4)

        xb_ref[:, :] = x_ref[:, :].astype(WIRE_DTYPE)
        scores = jnp.dot(xb_ref[:, :], rw_ref[:, :].astype(WIRE_DTYPE),
                         preferred_element_type=jnp.float32)
        r0 = idx_ref[:, 0:1]
        r1 = idx_ref[:, 1:2]
        eids = lax.broadcasted_iota(jnp.int32, (1, 64), 1)
        s0 = jnp.sum(scores * (r0 == eids).astype(jnp.float32),
                     axis=1, keepdims=True)
        s1 = jnp.sum(scores * (r1 == eids).astype(jnp.float32),
                     axis=1, keepdims=True)
        m = jnp.maximum(s0, s1)
        p0 = jnp.exp(s0 - m)
        p1 = jnp.exp(s1 - m)
        g0 = p0 / (p0 + p1)
        g1 = p1 / (p0 + p1)

        for j in range(E_LOCAL):
            e_glob = my * E_LOCAL + j
            gates_ref[:, j:j + 1] = (jnp.where(r0 == e_glob, g0, 0.0)
                                     + jnp.where(r1 == e_glob, g1, 0.0)) \
                .astype(WIRE_DTYPE)

        ewb_ref[:, :] = ew_ref[:, :].astype(WIRE_DTYPE)

        def group_half(g, col0):
            parts = []
            for zp in range(4):
                row0 = (4 * zp + g) * CHUNK
                xc = xb_ref[pl.ds(row0, CHUNK), :]
                parts.append(jnp.concatenate(
                    [xc * gates_ref[pl.ds(row0, CHUNK), j:j + 1]
                     for j in range(E_LOCAL)], axis=1))
            xcat = jnp.concatenate(parts, axis=0)
            return jnp.dot(xcat, ewb_ref[:, col0:col0 + H_HALF],
                           preferred_element_type=jnp.float32)

        def mk(src_ref, dst_ref, ssem, rsem, tgt):
            return pltpu.make_async_remote_copy(
                src_ref=src_ref, dst_ref=dst_ref, send_sem=ssem,
                recv_sem=rsem, device_id=(tgt,),
                device_id_type=pl.DeviceIdType.MESH)

        a_p = [mk(stap_ref, rap_ref.at[s], sap_sem, rap_sems.at[s], kp1)
               for s in range(3)]
        a_m = [mk(stam_ref, ram_ref.at[s], sam_sem, ram_sems.at[s], km1)
               for s in range(3)]
        b_p = [mk(stbp_ref, rbp_ref.at[s], sbp_sem, rbp_sems.at[s], zp1)
               for s in range(3)]
        b_m = [mk(stbm_ref, rbm_ref.at[s], sbm_sem, rbm_sems.at[s], zm1)
               for s in range(3)]

        f32 = jnp.float32

        stap_ref[:, :] = group_half(m4(k - 1), 0).astype(WIRE_DTYPE)
        a_p[0].start()
        stam_ref[:, :] = group_half(m4(k + 1), H_HALF).astype(WIRE_DTYPE)
        a_m[0].start()

        pr1 = group_half(m4(k + 2), 0).astype(WIRE_DTYPE)
        pl1 = group_half(m4(k + 2), H_HALF).astype(WIRE_DTYPE)
        a_p[0].wait_recv()
        a_p[0].wait_send()
        stap_ref[:, :] = rap_ref[0] + pr1
        a_p[1].start()
        a_m[0].wait_recv()
        a_m[0].wait_send()
        stam_ref[:, :] = ram_ref[0] + pl1
        a_m[1].start()

        pr2 = group_half(m4(k + 1), 0).astype(WIRE_DTYPE)
        pl2 = group_half(m4(k - 1), H_HALF).astype(WIRE_DTYPE)
        a_p[1].wait_recv()
        a_p[1].wait_send()
        stap_ref[:, :] = rap_ref[1] + pr2
        a_p[2].start()
        a_m[1].wait_recv()
        a_m[1].wait_send()
        stam_ref[:, :] = ram_ref[1] + pl2
        a_m[2].start()

        prf = group_half(k, 0).astype(WIRE_DTYPE)
        plf = group_half(k, H_HALF).astype(WIRE_DTYPE)

        a_p[2].wait_recv()
        par_ref[:, :] = rap_ref[2] + prf
        stbp_ref[:, :] = par_ref[pl.ds(m4(z - 1) * CHUNK, CHUNK), :]
        b_p[0].start()
        a_m[2].wait_recv()
        pal_ref[:, :] = ram_ref[2] + plf
        stbm_ref[:, :] = pal_ref[pl.ds(m4(z + 1) * CHUNK, CHUNK), :]
        b_m[0].start()
        for s in range(3):
            b_p[s].wait_recv()
            if s < 2:
                b_p[s].wait_send()
                stbp_ref[:, :] = (rbp_ref[s]
                                  + par_ref[pl.ds(m4(z - 2 - s) * CHUNK,
                                                  CHUNK), :])
                b_p[s + 1].start()
            else:
                out_ref[:, 0:H_HALF] = (
                    rbp_ref[s].astype(f32)
                    + par_ref[pl.ds(z * CHUNK, CHUNK), :].astype(f32))
            b_m[s].wait_recv()
            if s < 2:
                b_m[s].wait_send()
                stbm_ref[:, :] = (rbm_ref[s]
                                  + pal_ref[pl.ds(m4(z + 2 + s) * CHUNK,
                                                  CHUNK), :])
                b_m[s + 1].start()
            else:
                out_ref[:, H_HALF:D_OUT] = (
                    rbm_ref[s].astype(f32)
                    + pal_ref[pl.ds(z * CHUNK, CHUNK), :].astype(f32))

        a_p[2].wait_send()
        a_m[2].wait_send()
        b_p[2].wait_send()
        b_m[2].wait_send()

    return pl.pallas_call(
        body,
        out_shape=jax.ShapeDtypeStruct((CHUNK, D_OUT), jnp.float32),
        in_specs=[
            pl.BlockSpec(memory_space=pltpu.VMEM),
            pl.BlockSpec(memory_space=pltpu.VMEM),
            pl.BlockSpec(memory_space=pltpu.VMEM),
            pl.BlockSpec(memory_space=pltpu.VMEM),
        ],
        out_specs=pl.BlockSpec(memory_space=pltpu.VMEM),
        scratch_shapes=[
            pltpu.VMEM((N_TOK, E_LOCAL), WIRE_DTYPE),
            pltpu.VMEM((N_TOK, D_IN), WIRE_DTYPE),
            pltpu.VMEM((E_LOCAL * D_IN, D_OUT), WIRE_DTYPE),
            pltpu.VMEM((GROUP, H_HALF), WIRE_DTYPE),
            pltpu.VMEM((GROUP, H_HALF), WIRE_DTYPE),
            pltpu.VMEM((GROUP, H_HALF), WIRE_DTYPE),
            pltpu.VMEM((GROUP, H_HALF), WIRE_DTYPE),
            pltpu.VMEM((CHUNK, H_HALF), WIRE_DTYPE),
            pltpu.VMEM((CHUNK, H_HALF), WIRE_DTYPE),
            pltpu.VMEM((3, GROUP, H_HALF), WIRE_DTYPE),
            pltpu.VMEM((3, GROUP, H_HALF), WIRE_DTYPE),
            pltpu.VMEM((3, CHUNK, H_HALF), WIRE_DTYPE),
            pltpu.VMEM((3, CHUNK, H_HALF), WIRE_DTYPE),
            pltpu.SemaphoreType.DMA,
            pltpu.SemaphoreType.DMA,
            pltpu.SemaphoreType.DMA,
            pltpu.SemaphoreType.DMA,
            pltpu.SemaphoreType.DMA((3,)),
            pltpu.SemaphoreType.DMA((3,)),
            pltpu.SemaphoreType.DMA((3,)),
            pltpu.SemaphoreType.DMA((3,)),
        ],
        compiler_params=pltpu.CompilerParams(collective_id=0),
    )(x, router_W, route_idx, ew2)
